# Initial kernel scaffold; baseline (speedup 1.0000x reference)
import jax
import jax.numpy as jnp
from jax.experimental import pallas as pl


def kernel(position_ids, weight):
    raise NotImplementedError("write your pallas kernel here")


# speedup vs baseline: 1.6080x; 1.6080x over previous
import jax
import jax.numpy as jnp
from jax import lax
from jax.experimental import pallas as pl
from jax.experimental.pallas import tpu as pltpu
from jax.experimental.pallas import tpu_sc as plsc

B = 32768
D = 2048
NC = 2
NS = 16
NW = NC * NS
BPW = B // NW
C = 16
NCHUNK = BPW // C


def _emb_body(idx_hbm, table_hbm, out_hbm, idx_v, buf0, buf1,
              gsem0, gsem1, osem0, osem1):
    wid = lax.axis_index("s") * NC + lax.axis_index("c")
    base = wid * BPW
    pltpu.sync_copy(idx_hbm.at[pl.ds(base, BPW)], idx_v)

    def gather(g, buf, sem):
        return pltpu.make_async_copy(
            table_hbm.at[idx_v.at[pl.ds(g * C, C)]], buf, sem)

    def store(g, buf, sem):
        return pltpu.make_async_copy(
            buf, out_hbm.at[pl.ds(base + g * C, C)], sem)

    bufs = (buf0, buf1)
    gsems = (gsem0, gsem1)
    osems = (osem0, osem1)

    gather(0, buf0, gsem0).start()
    gather(1, buf1, gsem1).start()

    def step(h, last):
        for b in range(2):
            g = 2 * h + b
            gather(g, bufs[b], gsems[b]).wait()
            store(g, bufs[b], osems[b]).start()
            store(g, bufs[b], osems[b]).wait()
            if not last:
                gather(g + 2, bufs[b], gsems[b]).start()

    def body(h, carry):
        step(h, last=False)
        return carry

    lax.fori_loop(0, NCHUNK // 2 - 1, body, 0)
    step(NCHUNK // 2 - 1, last=True)


_emb = pl.kernel(
    _emb_body,
    out_type=jax.ShapeDtypeStruct((B, D), jnp.float32),
    mesh=plsc.VectorSubcoreMesh(core_axis_name="c", subcore_axis_name="s"),
    scratch_types=[
        pltpu.VMEM((BPW,), jnp.int32),
        pltpu.VMEM((C, D), jnp.float32),
        pltpu.VMEM((C, D), jnp.float32),
        pltpu.SemaphoreType.DMA,
        pltpu.SemaphoreType.DMA,
        pltpu.SemaphoreType.DMA,
        pltpu.SemaphoreType.DMA,
    ],
)


def kernel(position_ids, weight):
    idx = position_ids.reshape(-1).astype(jnp.int32)
    out = _emb(idx, weight)
    return out.reshape(position_ids.shape + (weight.shape[1],))
